# Initial kernel scaffold; baseline (speedup 1.0000x reference)
import jax
import jax.numpy as jnp
from jax.experimental import pallas as pl


def kernel(X, C, S):
    raise NotImplementedError("write your pallas kernel here")


# speedup vs baseline: 85.5786x; 85.5786x over previous
import numpy as np
import jax
import jax.numpy as jnp
from jax.experimental import pallas as pl
from jax.experimental.pallas import tpu as pltpu

_NUM_AA = 20
_NUM_SC = 10
_NUM_CHI = 4
_EPS = 1e-06

_trng = np.random.RandomState(0)
_NATOMS = _trng.randint(5, 15, size=_NUM_AA)
_NCHI = _trng.randint(0, 5, size=_NUM_AA)
_CHI_SETS = _trng.randint(0, 14, size=(_NUM_AA, _NUM_CHI, 4)).astype(np.int64)
_ZTAB = np.zeros((3, _NUM_SC, _NUM_AA), dtype=np.float32)
_ZTAB[0] = _trng.uniform(1.3, 1.6, size=(_NUM_SC, _NUM_AA))
_ZTAB[1] = _trng.uniform(90.0, 120.0, size=(_NUM_SC, _NUM_AA)) * np.pi / 180.0
_ZTAB[2] = _trng.uniform(-180.0, 180.0, size=(_NUM_SC, _NUM_AA)) * np.pi / 180.0
_PARENTS = np.zeros((3, _NUM_SC, _NUM_AA), dtype=np.int64)
for _j in range(_NUM_SC):
    _PARENTS[:, _j, :] = _trng.randint(0, 4 + _j, size=(3, _NUM_AA))
_CHI_IX = 10 * np.ones((_NUM_CHI, _NUM_AA), dtype=np.int64)
for _i in range(_NUM_AA):
    _perm = _trng.permutation(_NUM_SC)
    for _j in range(min(_NUM_CHI, int(_NCHI[_i]))):
        _CHI_IX[_j, _i] = _perm[_j]

_CHI_FLAT = _CHI_SETS.reshape(_NUM_AA, 16).astype(np.int32)
_PAR = np.transpose(_PARENTS, (2, 1, 0)).astype(np.int32)
_SRC = -np.ones((_NUM_AA, _NUM_SC), dtype=np.int32)
for _aa in range(_NUM_AA):
    for _k in range(_NUM_CHI):
        _jj = int(_CHI_IX[_k, _aa])
        if _jj < _NUM_SC:
            _SRC[_aa, _jj] = _k
_LTAB = _ZTAB[0].T.copy()
_ANG = _ZTAB[1].T.copy()
_DIH = _ZTAB[2].T.copy()

_F1 = np.float32(1.0)
_F0 = np.float32(0.0)


def _compute_block(xcols, Sv, Cv, cA, sA, cD0, sD0):
    f32 = jnp.float32
    m = [Sv == aa for aa in range(_NUM_AA)]

    def selv(vals):
        acc = vals[_NUM_AA - 1]
        for aa in range(_NUM_AA - 2, -1, -1):
            acc = jnp.where(m[aa], vals[aa], acc)
        return acc

    cposf = jnp.where(Cv > 0, _F1, _F0)

    def vsub(p, q):
        return [p[0] - q[0], p[1] - q[1], p[2] - q[2]]

    def vnormed(v):
        s = ((v[0] * v[0] + v[1] * v[1]) + v[2] * v[2]) + f32(_EPS)
        d = jnp.sqrt(s)
        return [v[0] / d, v[1] / d, v[2] / d]

    def vcross(a, b):
        return [a[1] * b[2] - a[2] * b[1],
                a[2] * b[0] - a[0] * b[2],
                a[0] * b[1] - a[1] * b[0]]

    def vdot(a, b):
        s = (a[0] * b[0] + a[1] * b[1]) + a[2] * b[2]
        return jnp.where(s == _F0, _F0, s)

    nchif = selv([f32(float(_NCHI[aa])) for aa in range(_NUM_AA)])
    per_chi = nchif * cposf
    cos_chi, sin_chi = [], []
    for k in range(_NUM_CHI):
        pts = []
        for a in range(4):
            pts.append([selv([xcols[3 * int(_CHI_FLAT[aa, 4 * k + a]) + c]
                              for aa in range(_NUM_AA)]) for c in range(3)])
        X1, X2, X3, X4 = pts
        u1 = vnormed(vsub(X2, X1))
        u2 = vnormed(vsub(X3, X2))
        u3 = vnormed(vsub(X4, X3))
        n1 = vnormed(vcross(u1, u2))
        n2 = vnormed(vcross(u2, u3))
        cos_d = vdot(n1, n2)
        sin_d = vdot(u2, vcross(n1, n2))
        chi = jnp.arctan2(sin_d, cos_d)
        mk = jnp.where(f32(float(k)) < per_chi, _F1, _F0)
        chim = chi * mk
        cos_chi.append(jnp.cos(chim))
        sin_chi.append(jnp.sin(chim))

    full = [xcols[c] for c in range(12)] + [None] * 30
    for i in range(_NUM_SC):
        par = []
        for t in range(3):
            par.append([selv([full[3 * int(_PAR[aa, i, t]) + c]
                              for aa in range(_NUM_AA)]) for c in range(3)])
        P1, P2, P3 = par
        Li = selv([f32(float(_LTAB[aa, i])) for aa in range(_NUM_AA)])
        cAi = selv([cA[aa, i] for aa in range(_NUM_AA)])
        sAi = selv([sA[aa, i] for aa in range(_NUM_AA)])
        cDi = selv([cD0[aa, i] for aa in range(_NUM_AA)])
        sDi = selv([sD0[aa, i] for aa in range(_NUM_AA)])
        for k in range(_NUM_CHI):
            aas = [aa for aa in range(_NUM_AA) if _SRC[aa, i] == k]
            if not aas:
                continue
            sm = m[aas[0]]
            for aa in aas[1:]:
                sm = jnp.logical_or(sm, m[aa])
            cDi = jnp.where(sm, cos_chi[k], cDi)
            sDi = jnp.where(sm, sin_chi[k], sDi)
        e1 = vnormed(vsub(P3, P2))
        e3 = vnormed(vcross(vsub(P1, P2), e1))
        e2 = vcross(e3, e1)
        ncA = -cAi
        sAcD = sAi * cDi
        sAsD = sAi * sDi
        for c in range(3):
            u_c = (ncA * e1[c] + sAcD * e2[c]) + sAsD * e3[c]
            full[12 + 3 * i + c] = _F0 + (P3[c] + Li * u_c)

    natf = selv([f32(float(_NATOMS[aa])) for aa in range(_NUM_AA)])
    per_at = natf * cposf
    out = []
    for a in range(14):
        mk = jnp.where(f32(float(a)) < per_at, _F1, _F0)
        for c in range(3):
            out.append(mk * full[3 * a + c])
    return out


def _pallas_body(x_ref, s_ref, c_ref, cA_ref, sA_ref, cD_ref, sD_ref, o_ref):
    xcols = [x_ref[c] for c in range(42)]
    outs = _compute_block(xcols, s_ref[...], c_ref[...],
                          cA_ref, sA_ref, cD_ref, sD_ref)
    for c in range(42):
        o_ref[c] = outs[c]


def kernel(X, C, S):
    nb, nr = S.shape
    R = nb * nr
    assert R % 1024 == 0
    nrow = R // 128
    grid = nrow // 8

    Xt = X.reshape(R, 42).T.reshape(42, nrow, 128)
    S2 = S.reshape(nrow, 128).astype(jnp.int32)
    C2 = C.reshape(nrow, 128).astype(jnp.int32)

    ang = jnp.asarray(_ANG)
    dih = jnp.asarray(_DIH)
    cA = jnp.cos(ang)
    sA = jnp.sin(ang)
    cD0 = jnp.cos(dih)
    sD0 = jnp.sin(dih)

    smem_spec = pl.BlockSpec((_NUM_AA, _NUM_SC), lambda g: (0, 0),
                             memory_space=pltpu.SMEM)
    out3 = pl.pallas_call(
        _pallas_body,
        grid=(grid,),
        in_specs=[
            pl.BlockSpec((42, 8, 128), lambda g: (0, g, 0)),
            pl.BlockSpec((8, 128), lambda g: (g, 0)),
            pl.BlockSpec((8, 128), lambda g: (g, 0)),
            smem_spec, smem_spec, smem_spec, smem_spec,
        ],
        out_specs=pl.BlockSpec((42, 8, 128), lambda g: (0, g, 0)),
        out_shape=jax.ShapeDtypeStruct((42, nrow, 128), jnp.float32),
    )(Xt, S2, C2, cA, sA, cD0, sD0)

    return out3.reshape(42, R).T.reshape(nb, nr, 14, 3)


# speedup vs baseline: 85.7016x; 1.0014x over previous
import numpy as np
import jax
import jax.numpy as jnp
from jax.experimental import pallas as pl
from jax.experimental.pallas import tpu as pltpu

_NUM_AA = 20
_NUM_SC = 10
_NUM_CHI = 4
_EPS = 1e-06

_trng = np.random.RandomState(0)
_NATOMS = _trng.randint(5, 15, size=_NUM_AA)
_NCHI = _trng.randint(0, 5, size=_NUM_AA)
_CHI_SETS = _trng.randint(0, 14, size=(_NUM_AA, _NUM_CHI, 4)).astype(np.int64)
_ZTAB = np.zeros((3, _NUM_SC, _NUM_AA), dtype=np.float32)
_ZTAB[0] = _trng.uniform(1.3, 1.6, size=(_NUM_SC, _NUM_AA))
_ZTAB[1] = _trng.uniform(90.0, 120.0, size=(_NUM_SC, _NUM_AA)) * np.pi / 180.0
_ZTAB[2] = _trng.uniform(-180.0, 180.0, size=(_NUM_SC, _NUM_AA)) * np.pi / 180.0
_PARENTS = np.zeros((3, _NUM_SC, _NUM_AA), dtype=np.int64)
for _j in range(_NUM_SC):
    _PARENTS[:, _j, :] = _trng.randint(0, 4 + _j, size=(3, _NUM_AA))
_CHI_IX = 10 * np.ones((_NUM_CHI, _NUM_AA), dtype=np.int64)
for _i in range(_NUM_AA):
    _perm = _trng.permutation(_NUM_SC)
    for _j in range(min(_NUM_CHI, int(_NCHI[_i]))):
        _CHI_IX[_j, _i] = _perm[_j]

_CHI_FLAT = _CHI_SETS.reshape(_NUM_AA, 16).astype(np.int32)
_PAR = np.transpose(_PARENTS, (2, 1, 0)).astype(np.int32)
_SRC = -np.ones((_NUM_AA, _NUM_SC), dtype=np.int32)
for _aa in range(_NUM_AA):
    for _k in range(_NUM_CHI):
        _jj = int(_CHI_IX[_k, _aa])
        if _jj < _NUM_SC:
            _SRC[_aa, _jj] = _k
_LTAB = _ZTAB[0].T.copy()
_ANG = _ZTAB[1].T.copy()
_DIH = _ZTAB[2].T.copy()

_F1 = np.float32(1.0)
_F0 = np.float32(0.0)


def _compute_block(xcols, Sv, Cv, cA, sA, cD0, sD0):
    f32 = jnp.float32
    m = [Sv == aa for aa in range(_NUM_AA)]

    def selv(vals):
        acc = vals[_NUM_AA - 1]
        for aa in range(_NUM_AA - 2, -1, -1):
            acc = jnp.where(m[aa], vals[aa], acc)
        return acc

    _coord_cache = {}

    def sel_cols(src, cols, tag):
        key = (tag, tuple(cols))
        r = _coord_cache.get(key)
        if r is None:
            r = selv([src[col] for col in cols])
            _coord_cache[key] = r
        return r

    cposf = jnp.where(Cv > 0, _F1, _F0)

    def vsub(p, q):
        return [p[0] - q[0], p[1] - q[1], p[2] - q[2]]

    def vnormed(v):
        s = ((v[0] * v[0] + v[1] * v[1]) + v[2] * v[2]) + f32(_EPS)
        d = jnp.sqrt(s)
        return [v[0] / d, v[1] / d, v[2] / d]

    def vcross(a, b):
        return [a[1] * b[2] - a[2] * b[1],
                a[2] * b[0] - a[0] * b[2],
                a[0] * b[1] - a[1] * b[0]]

    def vdot(a, b):
        s = (a[0] * b[0] + a[1] * b[1]) + a[2] * b[2]
        return jnp.where(s == _F0, _F0, s)

    nchif = selv([f32(float(_NCHI[aa])) for aa in range(_NUM_AA)])
    per_chi = nchif * cposf
    cos_chi, sin_chi = [], []
    for k in range(_NUM_CHI):
        pts = []
        for a in range(4):
            pts.append([sel_cols(xcols,
                                 [3 * int(_CHI_FLAT[aa, 4 * k + a]) + c
                                  for aa in range(_NUM_AA)], 'x')
                        for c in range(3)])
        X1, X2, X3, X4 = pts
        u1 = vnormed(vsub(X2, X1))
        u2 = vnormed(vsub(X3, X2))
        u3 = vnormed(vsub(X4, X3))
        n1 = vnormed(vcross(u1, u2))
        n2 = vnormed(vcross(u2, u3))
        cos_d = vdot(n1, n2)
        sin_d = vdot(u2, vcross(n1, n2))
        chi = jnp.arctan2(sin_d, cos_d)
        mk = jnp.where(f32(float(k)) < per_chi, _F1, _F0)
        chim = chi * mk
        cos_chi.append(jnp.cos(chim))
        sin_chi.append(jnp.sin(chim))

    full = [xcols[c] for c in range(12)] + [None] * 30
    for i in range(_NUM_SC):
        par = []
        for t in range(3):
            par.append([sel_cols(full,
                                 [3 * int(_PAR[aa, i, t]) + c
                                  for aa in range(_NUM_AA)], 'f')
                        for c in range(3)])
        P1, P2, P3 = par
        Li = selv([f32(float(_LTAB[aa, i])) for aa in range(_NUM_AA)])
        cAi = selv([cA[aa, i] for aa in range(_NUM_AA)])
        sAi = selv([sA[aa, i] for aa in range(_NUM_AA)])
        cDi = selv([cD0[aa, i] for aa in range(_NUM_AA)])
        sDi = selv([sD0[aa, i] for aa in range(_NUM_AA)])
        for k in range(_NUM_CHI):
            aas = [aa for aa in range(_NUM_AA) if _SRC[aa, i] == k]
            if not aas:
                continue
            sm = m[aas[0]]
            for aa in aas[1:]:
                sm = jnp.logical_or(sm, m[aa])
            cDi = jnp.where(sm, cos_chi[k], cDi)
            sDi = jnp.where(sm, sin_chi[k], sDi)
        e1 = vnormed(vsub(P3, P2))
        e3 = vnormed(vcross(vsub(P1, P2), e1))
        e2 = vcross(e3, e1)
        ncA = -cAi
        sAcD = sAi * cDi
        sAsD = sAi * sDi
        for c in range(3):
            u_c = (ncA * e1[c] + sAcD * e2[c]) + sAsD * e3[c]
            full[12 + 3 * i + c] = _F0 + (P3[c] + Li * u_c)

    natf = selv([f32(float(_NATOMS[aa])) for aa in range(_NUM_AA)])
    per_at = natf * cposf
    out = []
    for a in range(14):
        mk = jnp.where(f32(float(a)) < per_at, _F1, _F0)
        for c in range(3):
            out.append(mk * full[3 * a + c])
    return out


def _pallas_body(x_ref, s_ref, c_ref, cA_ref, sA_ref, cD_ref, sD_ref, o_ref):
    xcols = [x_ref[c] for c in range(42)]
    outs = _compute_block(xcols, s_ref[...], c_ref[...],
                          cA_ref, sA_ref, cD_ref, sD_ref)
    for c in range(42):
        o_ref[c] = outs[c]


def kernel(X, C, S):
    nb, nr = S.shape
    R = nb * nr
    assert R % 1024 == 0
    nrow = R // 128
    grid = nrow // 8

    Xt = X.reshape(R, 42).T.reshape(42, nrow, 128)
    S2 = S.reshape(nrow, 128).astype(jnp.int32)
    C2 = C.reshape(nrow, 128).astype(jnp.int32)

    ang = jnp.asarray(_ANG)
    dih = jnp.asarray(_DIH)
    cA = jnp.cos(ang)
    sA = jnp.sin(ang)
    cD0 = jnp.cos(dih)
    sD0 = jnp.sin(dih)

    smem_spec = pl.BlockSpec((_NUM_AA, _NUM_SC), lambda g: (0, 0),
                             memory_space=pltpu.SMEM)
    out3 = pl.pallas_call(
        _pallas_body,
        grid=(grid,),
        in_specs=[
            pl.BlockSpec((42, 8, 128), lambda g: (0, g, 0)),
            pl.BlockSpec((8, 128), lambda g: (g, 0)),
            pl.BlockSpec((8, 128), lambda g: (g, 0)),
            smem_spec, smem_spec, smem_spec, smem_spec,
        ],
        out_specs=pl.BlockSpec((42, 8, 128), lambda g: (0, g, 0)),
        out_shape=jax.ShapeDtypeStruct((42, nrow, 128), jnp.float32),
    )(Xt, S2, C2, cA, sA, cD0, sD0)

    return out3.reshape(42, R).T.reshape(nb, nr, 14, 3)


# speedup vs baseline: 89.1098x; 1.0398x over previous
import numpy as np
import jax
import jax.numpy as jnp
from jax.experimental import pallas as pl
from jax.experimental.pallas import tpu as pltpu

_NUM_AA = 20
_NUM_SC = 10
_NUM_CHI = 4
_EPS = 1e-06

_trng = np.random.RandomState(0)
_NATOMS = _trng.randint(5, 15, size=_NUM_AA)
_NCHI = _trng.randint(0, 5, size=_NUM_AA)
_CHI_SETS = _trng.randint(0, 14, size=(_NUM_AA, _NUM_CHI, 4)).astype(np.int64)
_ZTAB = np.zeros((3, _NUM_SC, _NUM_AA), dtype=np.float32)
_ZTAB[0] = _trng.uniform(1.3, 1.6, size=(_NUM_SC, _NUM_AA))
_ZTAB[1] = _trng.uniform(90.0, 120.0, size=(_NUM_SC, _NUM_AA)) * np.pi / 180.0
_ZTAB[2] = _trng.uniform(-180.0, 180.0, size=(_NUM_SC, _NUM_AA)) * np.pi / 180.0
_PARENTS = np.zeros((3, _NUM_SC, _NUM_AA), dtype=np.int64)
for _j in range(_NUM_SC):
    _PARENTS[:, _j, :] = _trng.randint(0, 4 + _j, size=(3, _NUM_AA))
_CHI_IX = 10 * np.ones((_NUM_CHI, _NUM_AA), dtype=np.int64)
for _i in range(_NUM_AA):
    _perm = _trng.permutation(_NUM_SC)
    for _j in range(min(_NUM_CHI, int(_NCHI[_i]))):
        _CHI_IX[_j, _i] = _perm[_j]

_CHI_FLAT = _CHI_SETS.reshape(_NUM_AA, 16).astype(np.int32)
_PAR = np.transpose(_PARENTS, (2, 1, 0)).astype(np.int32)
_SRC = -np.ones((_NUM_AA, _NUM_SC), dtype=np.int32)
for _aa in range(_NUM_AA):
    for _k in range(_NUM_CHI):
        _jj = int(_CHI_IX[_k, _aa])
        if _jj < _NUM_SC:
            _SRC[_aa, _jj] = _k
_LTAB = _ZTAB[0].T.copy()
_ANG = _ZTAB[1].T.copy()
_DIH = _ZTAB[2].T.copy()

_F1 = np.float32(1.0)
_F0 = np.float32(0.0)


def _compute_block(xcols, Sv, Cv, cA, sA, cD0, sD0):
    f32 = jnp.float32
    m = [Sv == aa for aa in range(_NUM_AA)]

    def selv(vals):
        acc = vals[_NUM_AA - 1]
        for aa in range(_NUM_AA - 2, -1, -1):
            acc = jnp.where(m[aa], vals[aa], acc)
        return acc

    _coord_cache = {}

    def sel_cols(src, cols, tag):
        key = (tag, tuple(cols))
        r = _coord_cache.get(key)
        if r is None:
            r = selv([src[col] for col in cols])
            _coord_cache[key] = r
        return r

    cposf = jnp.where(Cv > 0, _F1, _F0)

    def vsub(p, q):
        return [p[0] - q[0], p[1] - q[1], p[2] - q[2]]

    def vnormed(v):
        s = ((v[0] * v[0] + v[1] * v[1]) + v[2] * v[2]) + f32(_EPS)
        d = jnp.sqrt(s)
        return [v[0] / d, v[1] / d, v[2] / d]

    def vcross(a, b):
        return [a[1] * b[2] - a[2] * b[1],
                a[2] * b[0] - a[0] * b[2],
                a[0] * b[1] - a[1] * b[0]]

    def vdot(a, b):
        s = (a[0] * b[0] + a[1] * b[1]) + a[2] * b[2]
        return jnp.where(s == _F0, _F0, s)

    nchif = selv([f32(float(_NCHI[aa])) for aa in range(_NUM_AA)])
    per_chi = nchif * cposf
    cos_chi, sin_chi = [], []
    for k in range(_NUM_CHI):
        pts = []
        for a in range(4):
            pts.append([sel_cols(xcols,
                                 [3 * int(_CHI_FLAT[aa, 4 * k + a]) + c
                                  for aa in range(_NUM_AA)], 'x')
                        for c in range(3)])
        X1, X2, X3, X4 = pts
        u1 = vnormed(vsub(X2, X1))
        u2 = vnormed(vsub(X3, X2))
        u3 = vnormed(vsub(X4, X3))
        n1 = vnormed(vcross(u1, u2))
        n2 = vnormed(vcross(u2, u3))
        cos_d = vdot(n1, n2)
        sin_d = vdot(u2, vcross(n1, n2))
        chi = jnp.arctan2(sin_d, cos_d)
        mk = jnp.where(f32(float(k)) < per_chi, _F1, _F0)
        chim = chi * mk
        cos_chi.append(jnp.cos(chim))
        sin_chi.append(jnp.sin(chim))

    full = [xcols[c] for c in range(12)] + [None] * 30
    for i in range(_NUM_SC):
        par = []
        for t in range(3):
            par.append([sel_cols(full,
                                 [3 * int(_PAR[aa, i, t]) + c
                                  for aa in range(_NUM_AA)], 'f')
                        for c in range(3)])
        P1, P2, P3 = par
        Li = selv([f32(float(_LTAB[aa, i])) for aa in range(_NUM_AA)])
        cAi = selv([cA[aa, i] for aa in range(_NUM_AA)])
        sAi = selv([sA[aa, i] for aa in range(_NUM_AA)])
        cDi = selv([cD0[aa, i] for aa in range(_NUM_AA)])
        sDi = selv([sD0[aa, i] for aa in range(_NUM_AA)])
        for k in range(_NUM_CHI):
            aas = [aa for aa in range(_NUM_AA) if _SRC[aa, i] == k]
            if not aas:
                continue
            sm = m[aas[0]]
            for aa in aas[1:]:
                sm = jnp.logical_or(sm, m[aa])
            cDi = jnp.where(sm, cos_chi[k], cDi)
            sDi = jnp.where(sm, sin_chi[k], sDi)
        e1 = vnormed(vsub(P3, P2))
        e3 = vnormed(vcross(vsub(P1, P2), e1))
        e2 = vcross(e3, e1)
        ncA = -cAi
        sAcD = sAi * cDi
        sAsD = sAi * sDi
        for c in range(3):
            u_c = (ncA * e1[c] + sAcD * e2[c]) + sAsD * e3[c]
            full[12 + 3 * i + c] = _F0 + (P3[c] + Li * u_c)

    natf = selv([f32(float(_NATOMS[aa])) for aa in range(_NUM_AA)])
    per_at = natf * cposf
    out = []
    for a in range(14):
        mk = jnp.where(f32(float(a)) < per_at, _F1, _F0)
        for c in range(3):
            out.append(mk * full[3 * a + c])
    return out


def _pallas_body(x_ref, s_ref, c_ref, cA_ref, sA_ref, cD_ref, sD_ref, o_ref):
    xcols = [x_ref[c] for c in range(42)]
    outs = _compute_block(xcols, s_ref[...], c_ref[...],
                          cA_ref, sA_ref, cD_ref, sD_ref)
    for c in range(42):
        o_ref[c] = outs[c]


def kernel(X, C, S):
    nb, nr = S.shape
    R = nb * nr
    assert R % 2048 == 0
    nrow = R // 128
    grid = nrow // 16

    Xt = X.reshape(R, 42).T.reshape(42, nrow, 128)
    S2 = S.reshape(nrow, 128).astype(jnp.int32)
    C2 = C.reshape(nrow, 128).astype(jnp.int32)

    ang = jnp.asarray(_ANG)
    dih = jnp.asarray(_DIH)
    cA = jnp.cos(ang)
    sA = jnp.sin(ang)
    cD0 = jnp.cos(dih)
    sD0 = jnp.sin(dih)

    smem_spec = pl.BlockSpec((_NUM_AA, _NUM_SC), lambda g: (0, 0),
                             memory_space=pltpu.SMEM)
    out3 = pl.pallas_call(
        _pallas_body,
        grid=(grid,),
        in_specs=[
            pl.BlockSpec((42, 16, 128), lambda g: (0, g, 0)),
            pl.BlockSpec((16, 128), lambda g: (g, 0)),
            pl.BlockSpec((16, 128), lambda g: (g, 0)),
            smem_spec, smem_spec, smem_spec, smem_spec,
        ],
        out_specs=pl.BlockSpec((42, 16, 128), lambda g: (0, g, 0)),
        out_shape=jax.ShapeDtypeStruct((42, nrow, 128), jnp.float32),
    )(Xt, S2, C2, cA, sA, cD0, sD0)

    return out3.reshape(42, R).T.reshape(nb, nr, 14, 3)


# speedup vs baseline: 90.7144x; 1.0180x over previous
import numpy as np
import jax
import jax.numpy as jnp
from jax.experimental import pallas as pl
from jax.experimental.pallas import tpu as pltpu

_NUM_AA = 20
_NUM_SC = 10
_NUM_CHI = 4
_EPS = 1e-06

_trng = np.random.RandomState(0)
_NATOMS = _trng.randint(5, 15, size=_NUM_AA)
_NCHI = _trng.randint(0, 5, size=_NUM_AA)
_CHI_SETS = _trng.randint(0, 14, size=(_NUM_AA, _NUM_CHI, 4)).astype(np.int64)
_ZTAB = np.zeros((3, _NUM_SC, _NUM_AA), dtype=np.float32)
_ZTAB[0] = _trng.uniform(1.3, 1.6, size=(_NUM_SC, _NUM_AA))
_ZTAB[1] = _trng.uniform(90.0, 120.0, size=(_NUM_SC, _NUM_AA)) * np.pi / 180.0
_ZTAB[2] = _trng.uniform(-180.0, 180.0, size=(_NUM_SC, _NUM_AA)) * np.pi / 180.0
_PARENTS = np.zeros((3, _NUM_SC, _NUM_AA), dtype=np.int64)
for _j in range(_NUM_SC):
    _PARENTS[:, _j, :] = _trng.randint(0, 4 + _j, size=(3, _NUM_AA))
_CHI_IX = 10 * np.ones((_NUM_CHI, _NUM_AA), dtype=np.int64)
for _i in range(_NUM_AA):
    _perm = _trng.permutation(_NUM_SC)
    for _j in range(min(_NUM_CHI, int(_NCHI[_i]))):
        _CHI_IX[_j, _i] = _perm[_j]

_CHI_FLAT = _CHI_SETS.reshape(_NUM_AA, 16).astype(np.int32)
_PAR = np.transpose(_PARENTS, (2, 1, 0)).astype(np.int32)
_SRC = -np.ones((_NUM_AA, _NUM_SC), dtype=np.int32)
for _aa in range(_NUM_AA):
    for _k in range(_NUM_CHI):
        _jj = int(_CHI_IX[_k, _aa])
        if _jj < _NUM_SC:
            _SRC[_aa, _jj] = _k
_LTAB = _ZTAB[0].T.copy()
_ANG = _ZTAB[1].T.copy()
_DIH = _ZTAB[2].T.copy()

_F1 = np.float32(1.0)
_F0 = np.float32(0.0)


def _compute_block(xcols, Sv, Cv, cA, sA, cD0, sD0):
    f32 = jnp.float32
    m = [Sv == aa for aa in range(_NUM_AA)]

    def selv(vals):
        acc = vals[_NUM_AA - 1]
        for aa in range(_NUM_AA - 2, -1, -1):
            acc = jnp.where(m[aa], vals[aa], acc)
        return acc

    _coord_cache = {}

    def sel_cols(src, cols, tag):
        key = (tag, tuple(cols))
        r = _coord_cache.get(key)
        if r is None:
            r = selv([src[col] for col in cols])
            _coord_cache[key] = r
        return r

    cposf = jnp.where(Cv > 0, _F1, _F0)

    def vsub(p, q):
        return [p[0] - q[0], p[1] - q[1], p[2] - q[2]]

    def vnormed(v):
        s = ((v[0] * v[0] + v[1] * v[1]) + v[2] * v[2]) + f32(_EPS)
        d = jnp.sqrt(s)
        return [v[0] / d, v[1] / d, v[2] / d]

    def vcross(a, b):
        return [a[1] * b[2] - a[2] * b[1],
                a[2] * b[0] - a[0] * b[2],
                a[0] * b[1] - a[1] * b[0]]

    def vdot(a, b):
        s = (a[0] * b[0] + a[1] * b[1]) + a[2] * b[2]
        return jnp.where(s == _F0, _F0, s)

    nchif = selv([f32(float(_NCHI[aa])) for aa in range(_NUM_AA)])
    per_chi = nchif * cposf
    cos_chi, sin_chi = [], []
    for k in range(_NUM_CHI):
        pts = []
        for a in range(4):
            pts.append([sel_cols(xcols,
                                 [3 * int(_CHI_FLAT[aa, 4 * k + a]) + c
                                  for aa in range(_NUM_AA)], 'x')
                        for c in range(3)])
        X1, X2, X3, X4 = pts
        u1 = vnormed(vsub(X2, X1))
        u2 = vnormed(vsub(X3, X2))
        u3 = vnormed(vsub(X4, X3))
        n1 = vnormed(vcross(u1, u2))
        n2 = vnormed(vcross(u2, u3))
        cos_d = vdot(n1, n2)
        sin_d = vdot(u2, vcross(n1, n2))
        chi = jnp.arctan2(sin_d, cos_d)
        mk = jnp.where(f32(float(k)) < per_chi, _F1, _F0)
        chim = chi * mk
        cos_chi.append(jnp.cos(chim))
        sin_chi.append(jnp.sin(chim))

    full = [xcols[c] for c in range(12)] + [None] * 30
    for i in range(_NUM_SC):
        par = []
        for t in range(3):
            par.append([sel_cols(full,
                                 [3 * int(_PAR[aa, i, t]) + c
                                  for aa in range(_NUM_AA)], 'f')
                        for c in range(3)])
        P1, P2, P3 = par
        Li = selv([f32(float(_LTAB[aa, i])) for aa in range(_NUM_AA)])
        cAi = selv([cA[aa, i] for aa in range(_NUM_AA)])
        sAi = selv([sA[aa, i] for aa in range(_NUM_AA)])
        cDi = selv([cD0[aa, i] for aa in range(_NUM_AA)])
        sDi = selv([sD0[aa, i] for aa in range(_NUM_AA)])
        for k in range(_NUM_CHI):
            aas = [aa for aa in range(_NUM_AA) if _SRC[aa, i] == k]
            if not aas:
                continue
            sm = m[aas[0]]
            for aa in aas[1:]:
                sm = jnp.logical_or(sm, m[aa])
            cDi = jnp.where(sm, cos_chi[k], cDi)
            sDi = jnp.where(sm, sin_chi[k], sDi)
        e1 = vnormed(vsub(P3, P2))
        e3 = vnormed(vcross(vsub(P1, P2), e1))
        e2 = vcross(e3, e1)
        ncA = -cAi
        sAcD = sAi * cDi
        sAsD = sAi * sDi
        for c in range(3):
            u_c = (ncA * e1[c] + sAcD * e2[c]) + sAsD * e3[c]
            full[12 + 3 * i + c] = _F0 + (P3[c] + Li * u_c)

    natf = selv([f32(float(_NATOMS[aa])) for aa in range(_NUM_AA)])
    per_at = natf * cposf
    out = []
    for a in range(14):
        mk = jnp.where(f32(float(a)) < per_at, _F1, _F0)
        for c in range(3):
            out.append(mk * full[3 * a + c])
    return out


def _pallas_body(x_ref, s_ref, c_ref, cA_ref, sA_ref, cD_ref, sD_ref, o_ref):
    xcols = [x_ref[c] for c in range(42)]
    outs = _compute_block(xcols, s_ref[...], c_ref[...],
                          cA_ref, sA_ref, cD_ref, sD_ref)
    for c in range(42):
        o_ref[c] = outs[c]


def kernel(X, C, S):
    nb, nr = S.shape
    R = nb * nr
    assert R % 4096 == 0
    nrow = R // 128
    grid = nrow // 32

    Xt = X.reshape(R, 42).T.reshape(42, nrow, 128)
    S2 = S.reshape(nrow, 128).astype(jnp.int32)
    C2 = C.reshape(nrow, 128).astype(jnp.int32)

    ang = jnp.asarray(_ANG)
    dih = jnp.asarray(_DIH)
    cA = jnp.cos(ang)
    sA = jnp.sin(ang)
    cD0 = jnp.cos(dih)
    sD0 = jnp.sin(dih)

    smem_spec = pl.BlockSpec((_NUM_AA, _NUM_SC), lambda g: (0, 0),
                             memory_space=pltpu.SMEM)
    out3 = pl.pallas_call(
        _pallas_body,
        grid=(grid,),
        in_specs=[
            pl.BlockSpec((42, 32, 128), lambda g: (0, g, 0)),
            pl.BlockSpec((32, 128), lambda g: (g, 0)),
            pl.BlockSpec((32, 128), lambda g: (g, 0)),
            smem_spec, smem_spec, smem_spec, smem_spec,
        ],
        out_specs=pl.BlockSpec((42, 32, 128), lambda g: (0, g, 0)),
        out_shape=jax.ShapeDtypeStruct((42, nrow, 128), jnp.float32),
    )(Xt, S2, C2, cA, sA, cD0, sD0)

    return out3.reshape(42, R).T.reshape(nb, nr, 14, 3)
